# baseline (speedup 1.0000x reference)
import functools

import jax
import jax.numpy as jnp
from jax import lax
from jax.experimental import pallas as pl
from jax.experimental.pallas import tpu as pltpu
from jax.experimental.pallas import tpu_sc as plsc

V = 50000
K_EIG = 128
C_W = 128
T = 2000
NT = V // T

NNZ = 800000
NSUB = 16
NSLAB = 4
SLAB = C_W // NSLAB
EPS = NNZ // NSUB
E = 128
NB = EPS // E
ET = EPS - NB * E
ZB = 3120
ZS = 3200


def _sc_spmm(table, packed_main, packed_tail, zeros):
    mesh = plsc.VectorSubcoreMesh(core_axis_name="c", subcore_axis_name="s")

    @functools.partial(
        pl.kernel,
        out_type=(jax.ShapeDtypeStruct((NSLAB * V, SLAB), jnp.float32),
                  jax.ShapeDtypeStruct((NSLAB * V, SLAB), jnp.float32)),
        mesh=mesh,
        compiler_params=pltpu.CompilerParams(use_tc_tiling_on_sc=False),
        scratch_types=[
            pltpu.VMEM_SHARED((V, SLAB), jnp.float32),
            pltpu.VMEM((3, 3 * E), jnp.int32),
            pltpu.VMEM((E,), jnp.int32),
            pltpu.VMEM((E,), jnp.int32),
            pltpu.VMEM((E,), jnp.int32),
            pltpu.VMEM((E,), jnp.int32),
            pltpu.VMEM((E,), jnp.int32),
            pltpu.VMEM((E,), jnp.int32),
            pltpu.VMEM((3, E), jnp.float32),
            pltpu.VMEM((E, SLAB), jnp.float32),
            pltpu.VMEM((E, SLAB), jnp.float32),
            pltpu.VMEM((E, SLAB), jnp.float32),
            pltpu.VMEM((E, SLAB), jnp.float32),
            pltpu.VMEM((E, SLAB), jnp.float32),
            pltpu.VMEM((E, SLAB), jnp.float32),
            pltpu.VMEM((3 * ET,), jnp.int32),
            pltpu.VMEM((ET,), jnp.int32),
            pltpu.VMEM((ET,), jnp.int32),
        ] + [pltpu.SemaphoreType.DMA] * 9,
    )
    def k(table_h, pm_h, pt_h, zeros_h, out0_h, out1_h,
          shared, pbuf, rowbuf0, rowbuf1, rowbuf2,
          idxbuf0, idxbuf1, idxbuf2, valbuf,
          gbuf0, gbuf1, gbuf2, sbuf0, sbuf1, sbuf2,
          tpbuf, trowbuf, tidxbuf,
          psem0, psem1, psem2, gsem0, gsem1, gsem2, ssem0, ssem1, ssem2):
        cid = lax.axis_index("c")
        sid = lax.axis_index("s")
        rowbufs = (rowbuf0, rowbuf1, rowbuf2)
        idxbufs = (idxbuf0, idxbuf1, idxbuf2)
        gbufs = (gbuf0, gbuf1, gbuf2)
        sbufs = (sbuf0, sbuf1, sbuf2)
        psems = (psem0, psem1, psem2)
        gsems = (gsem0, gsem1, gsem2)
        ssems = (ssem0, ssem1, ssem2)

        def pk_start(bb, slot):
            pltpu.make_async_copy(pm_h.at[bb], pbuf.at[slot],
                                  psems[slot]).start()

        def pk_wait(bb, slot):
            pltpu.make_async_copy(pm_h.at[bb], pbuf.at[slot],
                                  psems[slot]).wait()

        def g_start(slot):
            pltpu.make_async_copy(table_h.at[idxbufs[slot]], gbufs[slot],
                                  gsems[slot]).start()

        def g_wait(slot):
            pltpu.make_async_copy(table_h.at[idxbufs[slot]], gbufs[slot],
                                  gsems[slot]).wait()

        def s_start(slot):
            pltpu.make_async_copy(sbufs[slot], shared.at[rowbufs[slot]],
                                  ssems[slot]).start(add=True)

        def s_wait(slot):
            pltpu.make_async_copy(sbufs[slot], shared.at[rowbufs[slot]],
                                  ssems[slot]).wait()

        def compute_bufs(slot, koff):
            @plsc.parallel_loop(0, E // 16, unroll=2)
            def _(j):
                sl = pl.ds(j * 16, 16)
                rowbufs[slot][sl] = pbuf[slot, pl.ds(j * 16, 16)]
                idxbufs[slot][sl] = pbuf[slot, pl.ds(E + j * 16, 16)] + koff
                valbuf[slot, sl] = lax.bitcast_convert_type(
                    pbuf[slot, pl.ds(2 * E + j * 16, 16)], jnp.float32)

        def scale(slot):
            @plsc.parallel_loop(0, E // 16, unroll=2)
            def _(j):
                val16 = valbuf[slot, pl.ds(j * 16, 16)]
                g = gbufs[slot]
                sb = sbufs[slot]
                for e16 in range(16):
                    v = val16[e16]
                    e = j * 16 + e16
                    sb[e, pl.ds(0, 16)] = g[e, pl.ds(0, 16)] * v
                    sb[e, pl.ds(16, 16)] = g[e, pl.ds(16, 16)] * v

        for i in range(4):
            p = 2 * i + cid
            o = p // NSLAB
            s = p % NSLAB
            pltpu.sync_copy(zeros_h.at[pl.ds(sid * ZB, ZS)],
                            shared.at[pl.ds(sid * ZB, ZS)])
            plsc.subcore_barrier()
            bbase = (o * NSUB + sid) * NB
            koff = s * V

            pk_start(bbase, 0)
            pk_start(bbase + 1, 1)
            pk_start(bbase + 2, 2)
            pk_wait(bbase, 0)
            compute_bufs(0, koff)
            pk_start(bbase + 3, 0)
            g_start(0)

            @pl.loop(0, NB, step=3)
            def _(b):
                for slot in (0, 1, 2):
                    bb = b + slot
                    ns = (slot + 1) % 3

                    @pl.when(bb + 1 < NB)
                    def _():
                        pk_wait(bbase + bb + 1, ns)

                        @pl.when(bb >= 2)
                        def _():
                            s_wait(ns)
                        compute_bufs(ns, koff)

                        @pl.when(bb + 4 < NB)
                        def _():
                            pk_start(bbase + bb + 4, ns)
                        g_start(ns)

                    g_wait(slot)
                    scale(slot)
                    s_start(slot)

            s_wait(0)
            s_wait(1)
            s_wait(2)

            ti = o * NSUB + sid
            pltpu.sync_copy(pt_h.at[ti], tpbuf)

            @pl.loop(0, ET // 16)
            def _(j):
                sl = pl.ds(j * 16, 16)
                trowbuf[sl] = tpbuf[pl.ds(j * 16, 16)]
                tidxbuf[sl] = tpbuf[pl.ds(ET + j * 16, 16)] + koff

            pltpu.sync_copy(table_h.at[tidxbuf], gbuf2.at[pl.ds(0, ET)])

            @plsc.parallel_loop(0, ET // 16, unroll=2)
            def _(j):
                val16 = lax.bitcast_convert_type(
                    tpbuf[pl.ds(2 * ET + j * 16, 16)], jnp.float32)
                for e16 in range(16):
                    v = val16[e16]
                    e = j * 16 + e16
                    sbuf2[e, pl.ds(0, 16)] = gbuf2[e, pl.ds(0, 16)] * v
                    sbuf2[e, pl.ds(16, 16)] = gbuf2[e, pl.ds(16, 16)] * v

            pltpu.sync_copy(sbuf2.at[pl.ds(0, ET)], shared.at[trowbuf],
                            add=True)

            plsc.subcore_barrier()

            @pl.when(cid == 0)
            def _():
                pltpu.sync_copy(shared.at[pl.ds(sid * ZB, ZS)],
                                out0_h.at[pl.ds(i * V + sid * ZB, ZS)])

            @pl.when(cid == 1)
            def _():
                pltpu.sync_copy(shared.at[pl.ds(sid * ZB, ZS)],
                                out1_h.at[pl.ds(i * V + sid * ZB, ZS)])

    return k(table, packed_main, packed_tail, zeros)


def _sweep0_body(inp_ref, mass_ref, evecs_ref, W0_ref, b0_ref,
                 x_ref, spec_ref, msum_ref, tbl_ref):
    t = pl.program_id(0)
    x_t = jnp.dot(inp_ref[...], W0_ref[...],
                  preferred_element_type=jnp.float32) + b0_ref[...]
    x_ref[...] = x_t
    for s in range(NSLAB):
        tbl_ref[s] = evecs_ref[:, SLAB * s:SLAB * (s + 1)]
    mz = x_t * mass_ref[...]
    contrib = lax.dot_general(evecs_ref[...], mz, (((0,), (0,)), ((), ())),
                              preferred_element_type=jnp.float32)
    msum_c = jnp.sum(mass_ref[...])

    @pl.when(t == 0)
    def _():
        spec_ref[...] = contrib
        msum_ref[...] = jnp.full((1, C_W), msum_c, jnp.float32)

    @pl.when(t != 0)
    def _():
        spec_ref[...] += contrib
        msum_ref[...] += msum_c


def _sweep_blk_body(last, x_ref, evecs_ref, a0_ref, a1_ref, mass_ref,
                    spec_ref, evals_ref, tb_ref, Are_ref, Aim_ref,
                    W1a_ref, W1b_ref, W1c_ref, b1_ref, W2_ref, b2_ref,
                    Wl_ref, out_x_ref, next_ref):
    t = pl.program_id(0)
    coefs = jnp.exp(-evals_ref[...] * tb_ref[...])
    S = coefs * spec_ref[...]
    x_t = x_ref[...]
    xd = jnp.dot(evecs_ref[...], S, preferred_element_type=jnp.float32)
    def _gsum(op):
        acc = None
        for c, aref in ((0, a0_ref), (1, a1_ref)):
            for j in range(NSLAB // 2):
                s = 2 * j + c
                d = jnp.dot(aref[op, j], S[SLAB * s:SLAB * (s + 1), :],
                            preferred_element_type=jnp.float32)
                acc = d if acc is None else acc + d
        return acc

    gX = _gsum(0)
    gY = _gsum(1)
    Are = Are_ref[...]
    Aim = Aim_ref[...]
    vre = (jnp.dot(gX, Are, preferred_element_type=jnp.float32)
           - jnp.dot(gY, Aim, preferred_element_type=jnp.float32))
    vim = (jnp.dot(gY, Are, preferred_element_type=jnp.float32)
           + jnp.dot(gX, Aim, preferred_element_type=jnp.float32))
    g = jnp.tanh(gX * vre + gY * vim)
    h = (jnp.dot(x_t, W1a_ref[...], preferred_element_type=jnp.float32)
         + jnp.dot(xd, W1b_ref[...], preferred_element_type=jnp.float32)
         + jnp.dot(g, W1c_ref[...], preferred_element_type=jnp.float32)
         + b1_ref[...])
    h = jnp.maximum(h, 0.0)
    xn = x_t + jnp.dot(h, W2_ref[...],
                       preferred_element_type=jnp.float32) + b2_ref[...]
    if not last:
        out_x_ref[...] = xn
        mz = xn * mass_ref[...]
        contrib = lax.dot_general(evecs_ref[...], mz,
                                  (((0,), (0,)), ((), ())),
                                  preferred_element_type=jnp.float32)
    else:
        y = jnp.dot(xn, Wl_ref[...], preferred_element_type=jnp.float32)
        contrib = jnp.sum(y * mass_ref[...], axis=0, keepdims=True)

    @pl.when(t == 0)
    def _():
        next_ref[...] = contrib

    @pl.when(t != 0)
    def _():
        next_ref[...] += contrib


def _full(shape):
    return pl.BlockSpec(shape, lambda t: tuple(0 for _ in shape))


def _vtile(c):
    return pl.BlockSpec((T, c), lambda t: (t, 0))


_CP = pltpu.CompilerParams(dimension_semantics=("arbitrary",))


@functools.partial(jax.jit, static_argnames=("last",))
def _run_block(last, x, evecs, gxe, gye, mass2, spec, evals_col, tb,
               Are, Aim, W1a, W1b, W1c, b1, W2, b2, Wl):
    out_shapes = (
        jax.ShapeDtypeStruct((V, C_W), jnp.float32),
        jax.ShapeDtypeStruct((1 if last else K_EIG, C_W), jnp.float32),
    )
    out_specs = (
        _vtile(C_W),
        pl.BlockSpec((1 if last else K_EIG, C_W), lambda t: (0, 0)),
    )
    return pl.pallas_call(
        functools.partial(_sweep_blk_body, last),
        grid=(NT,),
        in_specs=[
            _vtile(C_W), _vtile(C_W),
            pl.BlockSpec((2, NSLAB // 2, T, SLAB), lambda t: (0, 0, t, 0)),
            pl.BlockSpec((2, NSLAB // 2, T, SLAB), lambda t: (0, 0, t, 0)),
            _vtile(1),
            _full((K_EIG, C_W)), _full((K_EIG, 1)), _full((1, C_W)),
            _full((C_W, C_W)), _full((C_W, C_W)),
            _full((C_W, C_W)), _full((C_W, C_W)), _full((C_W, C_W)),
            _full((1, C_W)), _full((C_W, C_W)), _full((1, C_W)),
            _full((C_W, C_W)),
        ],
        out_specs=out_specs,
        out_shape=out_shapes,
        compiler_params=_CP,
    )(x, evecs, gxe, gye, mass2, spec, evals_col, tb, Are, Aim,
      W1a, W1b, W1c, b1, W2, b2, Wl)


def kernel(inputs, mass, evals, evecs, gradX_ind, gradX_val, gradY_ind,
           gradY_val, W0, b0, t_all, Are, Aim, W1, b1, W2, b2, Wl, bl):
    inp = inputs[0]
    mass2 = mass[:, None]
    evals_col = evals[:, None]

    x, spec, msum, tbl = pl.pallas_call(
        _sweep0_body,
        grid=(NT,),
        in_specs=[
            _vtile(inp.shape[1]), _vtile(1), _vtile(C_W),
            _full(W0.shape), _full((1, C_W)),
        ],
        out_specs=(
            _vtile(C_W),
            pl.BlockSpec((K_EIG, C_W), lambda t: (0, 0)),
            pl.BlockSpec((1, C_W), lambda t: (0, 0)),
            pl.BlockSpec((NSLAB, T, SLAB), lambda t: (0, t, 0)),
        ),
        out_shape=(
            jax.ShapeDtypeStruct((V, C_W), jnp.float32),
            jax.ShapeDtypeStruct((K_EIG, C_W), jnp.float32),
            jax.ShapeDtypeStruct((1, C_W), jnp.float32),
            jax.ShapeDtypeStruct((NSLAB, V, SLAB), jnp.float32),
        ),
        compiler_params=_CP,
    )(inp, mass2, evecs, W0, b0[None, :])

    rows_flat = jnp.concatenate([gradX_ind[0], gradY_ind[0]])
    cols_flat = jnp.concatenate([gradX_ind[1], gradY_ind[1]])
    vals_flat = jnp.concatenate([gradX_val, gradY_val])
    vbits = lax.bitcast_convert_type(vals_flat, jnp.int32)

    def _split(a):
        a = a.reshape(2, NSUB, EPS)
        return (a[:, :, :NB * E].reshape(2, NSUB, NB, E),
                a[:, :, NB * E:])

    rm, rt = _split(rows_flat)
    cm, ct = _split(cols_flat)
    vm, vt = _split(vbits)
    packed_main = jnp.stack([rm, cm, vm], axis=3).reshape(
        2 * NSUB * NB, 3 * E)
    packed_tail = jnp.stack([rt, ct, vt], axis=2).reshape(2 * NSUB, 3 * ET)
    zeros = jnp.zeros((V, SLAB), jnp.float32)
    sc0, sc1 = _sc_spmm(tbl.reshape(NSLAB * V, SLAB),
                        packed_main, packed_tail, zeros)
    gxe = sc0.reshape(2, NSLAB // 2, V, SLAB)
    gye = sc1.reshape(2, NSLAB // 2, V, SLAB)

    n_block = t_all.shape[0]
    for blk in range(n_block):
        last = blk == n_block - 1
        x, nxt = _run_block(
            last, x, evecs, gxe, gye, mass2, spec, evals_col,
            t_all[blk][None, :], Are[blk], Aim[blk],
            W1[blk, :C_W], W1[blk, C_W:2 * C_W], W1[blk, 2 * C_W:],
            b1[blk][None, :], W2[blk], b2[blk][None, :], Wl)
        spec = nxt
    out = nxt / msum[0, 0] + bl[None, :]
    return out


# speedup vs baseline: 1.0587x; 1.0587x over previous
import functools

import jax
import jax.numpy as jnp
from jax import lax
from jax.experimental import pallas as pl
from jax.experimental.pallas import tpu as pltpu
from jax.experimental.pallas import tpu_sc as plsc

V = 50000
K_EIG = 128
C_W = 128
T = 2000
NT = V // T

NNZ = 800000
NSUB = 16
NSLAB = 4
SLAB = C_W // NSLAB
EPS = NNZ // NSUB
E = 128
NB = EPS // E
ET = EPS - NB * E
ZB = 3120
ZS = 3200


def _sc_spmm(table, packed_main, packed_tail, zeros):
    mesh = plsc.VectorSubcoreMesh(core_axis_name="c", subcore_axis_name="s")

    @functools.partial(
        pl.kernel,
        out_type=(jax.ShapeDtypeStruct((NSLAB * V, SLAB), jnp.float32),
                  jax.ShapeDtypeStruct((NSLAB * V, SLAB), jnp.float32)),
        mesh=mesh,
        compiler_params=pltpu.CompilerParams(use_tc_tiling_on_sc=False),
        scratch_types=[
            pltpu.VMEM_SHARED((V, SLAB), jnp.float32),
            pltpu.VMEM((3, 3 * E), jnp.int32),
            pltpu.VMEM((E,), jnp.int32),
            pltpu.VMEM((E,), jnp.int32),
            pltpu.VMEM((E,), jnp.int32),
            pltpu.VMEM((E,), jnp.int32),
            pltpu.VMEM((E,), jnp.int32),
            pltpu.VMEM((E,), jnp.int32),
            pltpu.VMEM((3, E), jnp.float32),
            pltpu.VMEM((E, SLAB), jnp.float32),
            pltpu.VMEM((E, SLAB), jnp.float32),
            pltpu.VMEM((E, SLAB), jnp.float32),
            pltpu.VMEM((E, SLAB), jnp.float32),
            pltpu.VMEM((E, SLAB), jnp.float32),
            pltpu.VMEM((E, SLAB), jnp.float32),
            pltpu.VMEM((3 * ET,), jnp.int32),
            pltpu.VMEM((ET,), jnp.int32),
            pltpu.VMEM((ET,), jnp.int32),
        ] + [pltpu.SemaphoreType.DMA] * 9,
    )
    def k(table_h, pm_h, pt_h, zeros_h, out0_h, out1_h,
          shared, pbuf, rowbuf0, rowbuf1, rowbuf2,
          idxbuf0, idxbuf1, idxbuf2, valbuf,
          gbuf0, gbuf1, gbuf2, sbuf0, sbuf1, sbuf2,
          tpbuf, trowbuf, tidxbuf,
          psem0, psem1, psem2, gsem0, gsem1, gsem2, ssem0, ssem1, ssem2):
        cid = lax.axis_index("c")
        sid = lax.axis_index("s")
        rowbufs = (rowbuf0, rowbuf1, rowbuf2)
        idxbufs = (idxbuf0, idxbuf1, idxbuf2)
        gbufs = (gbuf0, gbuf1, gbuf2)
        sbufs = (sbuf0, sbuf1, sbuf2)
        psems = (psem0, psem1, psem2)
        gsems = (gsem0, gsem1, gsem2)
        ssems = (ssem0, ssem1, ssem2)

        def pk_start(bb, slot):
            pltpu.make_async_copy(pm_h.at[bb], pbuf.at[slot],
                                  psems[slot]).start()

        def pk_wait(bb, slot):
            pltpu.make_async_copy(pm_h.at[bb], pbuf.at[slot],
                                  psems[slot]).wait()

        def g_start(slot):
            pltpu.make_async_copy(table_h.at[idxbufs[slot]], gbufs[slot],
                                  gsems[slot]).start()

        def g_wait(slot):
            pltpu.make_async_copy(table_h.at[idxbufs[slot]], gbufs[slot],
                                  gsems[slot]).wait()

        def s_start(slot):
            pltpu.make_async_copy(gbufs[slot], shared.at[rowbufs[slot]],
                                  ssems[slot]).start(add=True)

        def s_wait(slot):
            pltpu.make_async_copy(gbufs[slot], shared.at[rowbufs[slot]],
                                  ssems[slot]).wait()

        def compute_bufs(slot, koff):
            @plsc.parallel_loop(0, E // 16, unroll=2)
            def _(j):
                sl = pl.ds(j * 16, 16)
                rowbufs[slot][sl] = pbuf[slot, pl.ds(j * 16, 16)]
                idxbufs[slot][sl] = pbuf[slot, pl.ds(E + j * 16, 16)] + koff
                valbuf[slot, sl] = lax.bitcast_convert_type(
                    pbuf[slot, pl.ds(2 * E + j * 16, 16)], jnp.float32)

        def scale(slot):
            @plsc.parallel_loop(0, E // 16, unroll=2)
            def _(j):
                val16 = valbuf[slot, pl.ds(j * 16, 16)]
                g = gbufs[slot]
                for e16 in range(16):
                    v = val16[e16]
                    e = j * 16 + e16
                    g[e, pl.ds(0, 16)] = g[e, pl.ds(0, 16)] * v
                    g[e, pl.ds(16, 16)] = g[e, pl.ds(16, 16)] * v

        for i in range(4):
            p = 2 * i + cid
            o = p // NSLAB
            s = p % NSLAB
            pltpu.sync_copy(zeros_h.at[pl.ds(sid * ZB, ZS)],
                            shared.at[pl.ds(sid * ZB, ZS)])
            plsc.subcore_barrier()
            bbase = (o * NSUB + sid) * NB
            koff = s * V

            pk_start(bbase, 0)
            pk_start(bbase + 1, 1)
            pk_start(bbase + 2, 2)
            pk_wait(bbase, 0)
            compute_bufs(0, koff)
            pk_start(bbase + 3, 0)
            g_start(0)

            @pl.loop(0, NB, step=3)
            def _(b):
                for slot in (0, 1, 2):
                    bb = b + slot
                    ns = (slot + 1) % 3

                    @pl.when(bb + 1 < NB)
                    def _():
                        pk_wait(bbase + bb + 1, ns)

                        @pl.when(bb >= 2)
                        def _():
                            s_wait(ns)
                        compute_bufs(ns, koff)

                        @pl.when(bb + 4 < NB)
                        def _():
                            pk_start(bbase + bb + 4, ns)
                        g_start(ns)

                    g_wait(slot)
                    scale(slot)
                    s_start(slot)

            s_wait(0)
            s_wait(1)
            s_wait(2)

            ti = o * NSUB + sid
            pltpu.sync_copy(pt_h.at[ti], tpbuf)

            @pl.loop(0, ET // 16)
            def _(j):
                sl = pl.ds(j * 16, 16)
                trowbuf[sl] = tpbuf[pl.ds(j * 16, 16)]
                tidxbuf[sl] = tpbuf[pl.ds(ET + j * 16, 16)] + koff

            pltpu.sync_copy(table_h.at[tidxbuf], gbuf2.at[pl.ds(0, ET)])

            @plsc.parallel_loop(0, ET // 16, unroll=2)
            def _(j):
                val16 = lax.bitcast_convert_type(
                    tpbuf[pl.ds(2 * ET + j * 16, 16)], jnp.float32)
                for e16 in range(16):
                    v = val16[e16]
                    e = j * 16 + e16
                    gbuf2[e, pl.ds(0, 16)] = gbuf2[e, pl.ds(0, 16)] * v
                    gbuf2[e, pl.ds(16, 16)] = gbuf2[e, pl.ds(16, 16)] * v

            pltpu.sync_copy(gbuf2.at[pl.ds(0, ET)], shared.at[trowbuf],
                            add=True)

            plsc.subcore_barrier()

            @pl.when(cid == 0)
            def _():
                pltpu.sync_copy(shared.at[pl.ds(sid * ZB, ZS)],
                                out0_h.at[pl.ds(i * V + sid * ZB, ZS)])

            @pl.when(cid == 1)
            def _():
                pltpu.sync_copy(shared.at[pl.ds(sid * ZB, ZS)],
                                out1_h.at[pl.ds(i * V + sid * ZB, ZS)])

    return k(table, packed_main, packed_tail, zeros)


def _sweep0_body(inp_ref, mass_ref, evecs_ref, W0_ref, b0_ref,
                 x_ref, spec_ref, msum_ref, tbl_ref):
    t = pl.program_id(0)
    x_t = jnp.dot(inp_ref[...], W0_ref[...],
                  preferred_element_type=jnp.float32) + b0_ref[...]
    x_ref[...] = x_t
    for s in range(NSLAB):
        tbl_ref[s] = evecs_ref[:, SLAB * s:SLAB * (s + 1)]
    mz = x_t * mass_ref[...]
    contrib = lax.dot_general(evecs_ref[...], mz, (((0,), (0,)), ((), ())),
                              preferred_element_type=jnp.float32)
    msum_c = jnp.sum(mass_ref[...])

    @pl.when(t == 0)
    def _():
        spec_ref[...] = contrib
        msum_ref[...] = jnp.full((1, C_W), msum_c, jnp.float32)

    @pl.when(t != 0)
    def _():
        spec_ref[...] += contrib
        msum_ref[...] += msum_c


def _sweep_blk_body(last, x_ref, evecs_ref, a0_ref, a1_ref, mass_ref,
                    spec_ref, evals_ref, tb_ref, Are_ref, Aim_ref,
                    W1a_ref, W1b_ref, W1c_ref, b1_ref, W2_ref, b2_ref,
                    Wl_ref, out_x_ref, next_ref):
    t = pl.program_id(0)
    coefs = jnp.exp(-evals_ref[...] * tb_ref[...])
    S = coefs * spec_ref[...]
    x_t = x_ref[...]
    xd = jnp.dot(evecs_ref[...], S, preferred_element_type=jnp.float32)
    def _gsum(op):
        acc = None
        for c, aref in ((0, a0_ref), (1, a1_ref)):
            for j in range(NSLAB // 2):
                s = 2 * j + c
                d = jnp.dot(aref[op, j], S[SLAB * s:SLAB * (s + 1), :],
                            preferred_element_type=jnp.float32)
                acc = d if acc is None else acc + d
        return acc

    gX = _gsum(0)
    gY = _gsum(1)
    Are = Are_ref[...]
    Aim = Aim_ref[...]
    vre = (jnp.dot(gX, Are, preferred_element_type=jnp.float32)
           - jnp.dot(gY, Aim, preferred_element_type=jnp.float32))
    vim = (jnp.dot(gY, Are, preferred_element_type=jnp.float32)
           + jnp.dot(gX, Aim, preferred_element_type=jnp.float32))
    g = jnp.tanh(gX * vre + gY * vim)
    h = (jnp.dot(x_t, W1a_ref[...], preferred_element_type=jnp.float32)
         + jnp.dot(xd, W1b_ref[...], preferred_element_type=jnp.float32)
         + jnp.dot(g, W1c_ref[...], preferred_element_type=jnp.float32)
         + b1_ref[...])
    h = jnp.maximum(h, 0.0)
    xn = x_t + jnp.dot(h, W2_ref[...],
                       preferred_element_type=jnp.float32) + b2_ref[...]
    if not last:
        out_x_ref[...] = xn
        mz = xn * mass_ref[...]
        contrib = lax.dot_general(evecs_ref[...], mz,
                                  (((0,), (0,)), ((), ())),
                                  preferred_element_type=jnp.float32)
    else:
        y = jnp.dot(xn, Wl_ref[...], preferred_element_type=jnp.float32)
        contrib = jnp.sum(y * mass_ref[...], axis=0, keepdims=True)

    @pl.when(t == 0)
    def _():
        next_ref[...] = contrib

    @pl.when(t != 0)
    def _():
        next_ref[...] += contrib


def _full(shape):
    return pl.BlockSpec(shape, lambda t: tuple(0 for _ in shape))


def _vtile(c):
    return pl.BlockSpec((T, c), lambda t: (t, 0))


_CP = pltpu.CompilerParams(dimension_semantics=("arbitrary",))


@functools.partial(jax.jit, static_argnames=("last",))
def _run_block(last, x, evecs, gxe, gye, mass2, spec, evals_col, tb,
               Are, Aim, W1a, W1b, W1c, b1, W2, b2, Wl):
    out_shapes = (
        jax.ShapeDtypeStruct((V, C_W), jnp.float32),
        jax.ShapeDtypeStruct((1 if last else K_EIG, C_W), jnp.float32),
    )
    out_specs = (
        _vtile(C_W),
        pl.BlockSpec((1 if last else K_EIG, C_W), lambda t: (0, 0)),
    )
    return pl.pallas_call(
        functools.partial(_sweep_blk_body, last),
        grid=(NT,),
        in_specs=[
            _vtile(C_W), _vtile(C_W),
            pl.BlockSpec((2, NSLAB // 2, T, SLAB), lambda t: (0, 0, t, 0)),
            pl.BlockSpec((2, NSLAB // 2, T, SLAB), lambda t: (0, 0, t, 0)),
            _vtile(1),
            _full((K_EIG, C_W)), _full((K_EIG, 1)), _full((1, C_W)),
            _full((C_W, C_W)), _full((C_W, C_W)),
            _full((C_W, C_W)), _full((C_W, C_W)), _full((C_W, C_W)),
            _full((1, C_W)), _full((C_W, C_W)), _full((1, C_W)),
            _full((C_W, C_W)),
        ],
        out_specs=out_specs,
        out_shape=out_shapes,
        compiler_params=_CP,
    )(x, evecs, gxe, gye, mass2, spec, evals_col, tb, Are, Aim,
      W1a, W1b, W1c, b1, W2, b2, Wl)


def kernel(inputs, mass, evals, evecs, gradX_ind, gradX_val, gradY_ind,
           gradY_val, W0, b0, t_all, Are, Aim, W1, b1, W2, b2, Wl, bl):
    inp = inputs[0]
    mass2 = mass[:, None]
    evals_col = evals[:, None]

    x, spec, msum, tbl = pl.pallas_call(
        _sweep0_body,
        grid=(NT,),
        in_specs=[
            _vtile(inp.shape[1]), _vtile(1), _vtile(C_W),
            _full(W0.shape), _full((1, C_W)),
        ],
        out_specs=(
            _vtile(C_W),
            pl.BlockSpec((K_EIG, C_W), lambda t: (0, 0)),
            pl.BlockSpec((1, C_W), lambda t: (0, 0)),
            pl.BlockSpec((NSLAB, T, SLAB), lambda t: (0, t, 0)),
        ),
        out_shape=(
            jax.ShapeDtypeStruct((V, C_W), jnp.float32),
            jax.ShapeDtypeStruct((K_EIG, C_W), jnp.float32),
            jax.ShapeDtypeStruct((1, C_W), jnp.float32),
            jax.ShapeDtypeStruct((NSLAB, V, SLAB), jnp.float32),
        ),
        compiler_params=_CP,
    )(inp, mass2, evecs, W0, b0[None, :])

    rows_flat = jnp.concatenate([gradX_ind[0], gradY_ind[0]])
    cols_flat = jnp.concatenate([gradX_ind[1], gradY_ind[1]])
    vals_flat = jnp.concatenate([gradX_val, gradY_val])
    vbits = lax.bitcast_convert_type(vals_flat, jnp.int32)

    def _split(a):
        a = a.reshape(2, NSUB, EPS)
        return (a[:, :, :NB * E].reshape(2, NSUB, NB, E),
                a[:, :, NB * E:])

    rm, rt = _split(rows_flat)
    cm, ct = _split(cols_flat)
    vm, vt = _split(vbits)
    packed_main = jnp.stack([rm, cm, vm], axis=3).reshape(
        2 * NSUB * NB, 3 * E)
    packed_tail = jnp.stack([rt, ct, vt], axis=2).reshape(2 * NSUB, 3 * ET)
    zeros = jnp.zeros((V, SLAB), jnp.float32)
    sc0, sc1 = _sc_spmm(tbl.reshape(NSLAB * V, SLAB),
                        packed_main, packed_tail, zeros)
    gxe = sc0.reshape(2, NSLAB // 2, V, SLAB)
    gye = sc1.reshape(2, NSLAB // 2, V, SLAB)

    n_block = t_all.shape[0]
    for blk in range(n_block):
        last = blk == n_block - 1
        x, nxt = _run_block(
            last, x, evecs, gxe, gye, mass2, spec, evals_col,
            t_all[blk][None, :], Are[blk], Aim[blk],
            W1[blk, :C_W], W1[blk, C_W:2 * C_W], W1[blk, 2 * C_W:],
            b1[blk][None, :], W2[blk], b2[blk][None, :], Wl)
        spec = nxt
    out = nxt / msum[0, 0] + bl[None, :]
    return out
